# baseline (speedup 1.0000x reference)
import functools

import jax
import jax.numpy as jnp
from jax import lax
from jax.experimental import pallas as pl
from jax.experimental.pallas import tpu as pltpu
from jax.experimental.pallas import tpu_sc as plsc

N = 100000
E = 1600000
G = 64
NCLS = 10
RP = 8192
P = 13
NP = P * RP
NW = 32
CHUNK = E // NW
SUBA = 25088
SUBB = CHUNK - SUBA
BATCH = 512
E8 = E + NW * P * BATCH
ACCR = RP + 8
TROWS = RP // 16
ZR = 64

_SC_PARAMS = pltpu.CompilerParams(needs_layout_passes=False,
                                  use_tc_tiling_on_sc=False)


@functools.cache
def _mesh():
    return plsc.VectorSubcoreMesh(core_axis_name="c", subcore_axis_name="s",
                                  num_cores=2, num_subcores=16)


def _count_body(dst_hbm, cnt_hbm, dvm, cvm):
    c = lax.axis_index("c")
    s = lax.axis_index("s")
    w = 2 * s + c
    pltpu.sync_copy(dst_hbm.at[pl.ds(w * CHUNK, CHUNK)], dvm)
    lanes = lax.iota(jnp.int32, 16)

    def body(i, accs):
        d = dvm[pl.ds(i * 16, 16)]
        pid = lax.shift_right_logical(d, 13)
        return tuple(a + (pid == p).astype(jnp.int32)
                     for p, a in enumerate(accs))

    accs = lax.fori_loop(0, CHUNK // 16, body,
                         tuple(jnp.zeros((16,), jnp.int32)
                               for _ in range(P)))
    cvec = jnp.zeros((16,), jnp.int32)
    for p in range(P):
        cvec = jnp.where(lanes == p, jnp.sum(accs[p]), cvec)
    cvm[...] = cvec
    pltpu.sync_copy(cvm, cnt_hbm.at[pl.ds(w * 16, 16)])


@functools.cache
def _count_call():
    return pl.kernel(
        _count_body,
        out_type=jax.ShapeDtypeStruct((NW * 16,), jnp.int32),
        mesh=_mesh(),
        scratch_types=[pltpu.VMEM((CHUNK,), jnp.int32),
                       pltpu.VMEM((16,), jnp.int32)],
        compiler_params=_SC_PARAMS,
    )


def _scan_body(cnt_ref, st_ref, nb_ref):
    nbi = lax.shift_right_logical(cnt_ref[...] + (BATCH - 1), 9)
    nbf = nbi.astype(jnp.float32)
    r = lax.broadcasted_iota(jnp.int32, (NW * 16, NW * 16), 0)
    col = lax.broadcasted_iota(jnp.int32, (NW * 16, NW * 16), 1)
    tril = (col < r).astype(jnp.float32)
    st = jnp.dot(tril, nbf, preferred_element_type=jnp.float32)
    st_ref[...] = st.astype(jnp.int32)
    nb_ref[...] = nbi


_scan_call = pl.pallas_call(
    _scan_body,
    out_shape=[jax.ShapeDtypeStruct((NW * 16, 1), jnp.int32),
               jax.ShapeDtypeStruct((NW * 16, 1), jnp.int32)],
)


def _bucket_body(src_hbm, dst_hbm, st_hbm, bsrc_hbm, bldst_hbm,
                 svm, dvm, bs, bl, stv, cur_ref, nfl_ref):
    c = lax.axis_index("c")
    s = lax.axis_index("s")
    w = 2 * s + c
    pltpu.sync_copy(st_hbm.at[pl.ds(w * 16, 16)], stv)
    for p in range(P):
        cur_ref[p] = jnp.int32(0)
        nfl_ref[p] = jnp.int32(0)

    PB = BATCH + 16

    def handle(p, sv, ld, m):
        cur = cur_ref[p]
        ps = plsc.cumsum(jnp.where(m, 1, 0))
        idx = p * PB + cur + ps - 1
        plsc.store_scatter(bs, [idx], sv, mask=m)
        plsc.store_scatter(bl, [idx], ld, mask=m)
        cur = cur + ps[15]
        cur_ref[p] = cur

        @pl.when(cur >= BATCH)
        def _():
            segw = (stv[...][p] + nfl_ref[p]) * BATCH
            pltpu.sync_copy(bs.at[pl.ds(p * PB, BATCH)],
                            bsrc_hbm.at[pl.ds(segw, BATCH)])
            pltpu.sync_copy(bl.at[pl.ds(p * PB, BATCH)],
                            bldst_hbm.at[pl.ds(segw, BATCH)])
            ts = bs[pl.ds(p * PB + BATCH, 16)]
            tl = bl[pl.ds(p * PB + BATCH, 16)]
            bs[pl.ds(p * PB, 16)] = ts
            bl[pl.ds(p * PB, 16)] = tl
            nfl_ref[p] = nfl_ref[p] + 1
            cur_ref[p] = cur - BATCH

    for half, sub in enumerate((SUBA, SUBB)):
        base = w * CHUNK + half * SUBA
        pltpu.sync_copy(src_hbm.at[pl.ds(base, sub)],
                        svm.at[pl.ds(0, sub)])
        pltpu.sync_copy(dst_hbm.at[pl.ds(base, sub)],
                        dvm.at[pl.ds(0, sub)])

        def body(i, _):
            sv = svm[pl.ds(i * 16, 16)]
            dv = dvm[pl.ds(i * 16, 16)]
            pid = lax.shift_right_logical(dv, 13)
            ld = jnp.bitwise_and(dv, RP - 1)
            for p in range(P):
                handle(p, sv, ld, pid == p)
            return 0

        lax.fori_loop(0, sub // 16, body, 0)

    lanes = lax.iota(jnp.int32, 16)
    for p in range(P):
        cur = cur_ref[p]
        for j in range(BATCH // 16):
            fill = (lanes + j * 16) >= cur
            ov_s = bs[pl.ds(p * PB + j * 16, 16)]
            ov_l = bl[pl.ds(p * PB + j * 16, 16)]
            bs[pl.ds(p * PB + j * 16, 16)] = jnp.where(fill, 0, ov_s)
            bl[pl.ds(p * PB + j * 16, 16)] = jnp.where(fill, RP + (j % 8),
                                                       ov_l)

        @pl.when(cur > 0)
        def _():
            segw = (stv[...][p] + nfl_ref[p]) * BATCH
            pltpu.sync_copy(bs.at[pl.ds(p * PB, BATCH)],
                            bsrc_hbm.at[pl.ds(segw, BATCH)])
            pltpu.sync_copy(bl.at[pl.ds(p * PB, BATCH)],
                            bldst_hbm.at[pl.ds(segw, BATCH)])


@functools.cache
def _bucket_call():
    return pl.kernel(
        _bucket_body,
        out_type=(jax.ShapeDtypeStruct((E8,), jnp.int32),
                  jax.ShapeDtypeStruct((E8,), jnp.int32)),
        mesh=_mesh(),
        scratch_types=[
            pltpu.VMEM((SUBA,), jnp.int32),
            pltpu.VMEM((SUBA,), jnp.int32),
            pltpu.VMEM((P * (BATCH + 16),), jnp.int32),
            pltpu.VMEM((P * (BATCH + 16),), jnp.int32),
            pltpu.VMEM((16,), jnp.int32),
            pltpu.SMEM((16,), jnp.int32),
            pltpu.SMEM((16,), jnp.int32),
        ],
        compiler_params=_SC_PARAMS,
    )


def _agg_inner(g_hbm, bsrc, bldst, out_hbm, lvm0, lvm1, svm0, svm1,
               rows0, rows1, zstage, stv, nbv, acc,
               siL0, siL1, siS0, siS1, sg0, sg1, deg_pass, SB):
    c = lax.axis_index("c")
    s = lax.axis_index("s")
    NSB = BATCH // SB

    for p in range(P):
        myrows = p * RP + s * TROWS
        if deg_pass:
            for k in range(TROWS // ZR):
                pltpu.sync_copy(zstage,
                                acc.at[pl.ds(s * TROWS + k * ZR, ZR)])
        else:
            @pl.when(c == 0)
            def _():
                pltpu.sync_copy(g_hbm.at[pl.ds(myrows, TROWS)],
                                acc.at[pl.ds(s * TROWS, TROWS)])

            @pl.when(c == 1)
            def _():
                for k in range(TROWS // ZR):
                    pltpu.sync_copy(zstage,
                                    acc.at[pl.ds(s * TROWS + k * ZR, ZR)])

        plsc.subcore_barrier()
        stb = stv[...][p]
        nbb = nbv[...][p] * NSB
        base = stb * BATCH

        def idx_start(jj, lvmk, svmk, semL, semS):
            off = base + jnp.minimum(jj, nbb - 1) * SB
            pltpu.async_copy(bldst.at[pl.ds(off, SB)], lvmk, semL)
            if not deg_pass:
                pltpu.async_copy(bsrc.at[pl.ds(off, SB)], svmk, semS)

        def idx_wait(lvmk, svmk, semL, semS):
            pltpu.make_async_copy(bldst.at[pl.ds(0, SB)], lvmk, semL).wait()
            if not deg_pass:
                pltpu.make_async_copy(bsrc.at[pl.ds(0, SB)], svmk,
                                      semS).wait()

        @pl.when(nbb > 0)
        def _():
            idx_start(0, lvm0, svm0, siL0, siS0)
            idx_start(1, lvm1, svm1, siL1, siS1)

        def body(j2, _):
            j0 = 2 * j2
            j1 = j0 + 1
            idx_wait(lvm0, svm0, siL0, siS0)
            if not deg_pass:
                d0 = pltpu.async_copy(g_hbm.at[svm0], rows0, sg0)
            idx_wait(lvm1, svm1, siL1, siS1)
            if not deg_pass:
                d1 = pltpu.async_copy(g_hbm.at[svm1], rows1, sg1)
                d0.wait()

            @pl.when(j0 < nbb)
            def _():
                pltpu.sync_copy(rows0, acc.at[lvm0], add=True)

            idx_start(j0 + 2, lvm0, svm0, siL0, siS0)
            if not deg_pass:
                d1.wait()

            @pl.when(j1 < nbb)
            def _():
                pltpu.sync_copy(rows1 if not deg_pass else rows0,
                                acc.at[lvm1], add=True)

            idx_start(j1 + 2, lvm1, svm1, siL1, siS1)
            return 0

        lax.fori_loop(0, (nbb + 1) // 2, body, 0)

        @pl.when(nbb > 0)
        def _():
            idx_wait(lvm0, svm0, siL0, siS0)
            idx_wait(lvm1, svm1, siL1, siS1)

        plsc.subcore_barrier()
        pltpu.sync_copy(acc.at[pl.ds(s * TROWS, TROWS)],
                        out_hbm.at[pl.ds(c * NP + myrows, TROWS)])
        plsc.subcore_barrier()


@functools.cache
def _make_agg(w_feat):
    SB = 128 if w_feat == 128 else (256 if w_feat == 64 else BATCH)
    scratch = [
        pltpu.VMEM((SB,), jnp.int32),
        pltpu.VMEM((SB,), jnp.int32),
        pltpu.VMEM((SB,), jnp.int32),
        pltpu.VMEM((SB,), jnp.int32),
        pltpu.VMEM((SB, w_feat), jnp.float32),
        pltpu.VMEM((SB, w_feat), jnp.float32),
        pltpu.VMEM((ZR, w_feat), jnp.float32),
        pltpu.VMEM((16,), jnp.int32),
        pltpu.VMEM((16,), jnp.int32),
        pltpu.VMEM_SHARED((ACCR, w_feat), jnp.float32),
        pltpu.SemaphoreType.DMA,
        pltpu.SemaphoreType.DMA,
        pltpu.SemaphoreType.DMA,
        pltpu.SemaphoreType.DMA,
        pltpu.SemaphoreType.DMA,
        pltpu.SemaphoreType.DMA,
    ]
    out = jax.ShapeDtypeStruct((2 * NP, w_feat), jnp.float32)

    def body(g_hbm, z_hbm, bsrc, bldst, st_hbm, nb_hbm, out_hbm,
             lvm0, lvm1, svm0, svm1, rows0, rows1, zstage, stv, nbv, acc,
             siL0, siL1, siS0, siS1, sg0, sg1):
        w = 2 * lax.axis_index("s") + lax.axis_index("c")
        pltpu.sync_copy(st_hbm.at[pl.ds(w * 16, 16)], stv)
        pltpu.sync_copy(nb_hbm.at[pl.ds(w * 16, 16)], nbv)
        pltpu.sync_copy(z_hbm, zstage)
        _agg_inner(g_hbm, bsrc, bldst, out_hbm, lvm0, lvm1, svm0, svm1,
                   rows0, rows1, zstage, stv, nbv, acc,
                   siL0, siL1, siS0, siS1, sg0, sg1, False, SB)

    return pl.kernel(body, out_type=out, mesh=_mesh(),
                     scratch_types=scratch, compiler_params=_SC_PARAMS)


@functools.cache
def _make_deg():
    scratch = [
        pltpu.VMEM((BATCH,), jnp.int32),
        pltpu.VMEM((BATCH,), jnp.int32),
        pltpu.VMEM((BATCH, 16), jnp.float32),
        pltpu.VMEM((ZR, 16), jnp.float32),
        pltpu.VMEM((16,), jnp.int32),
        pltpu.VMEM((16,), jnp.int32),
        pltpu.VMEM_SHARED((ACCR, 16), jnp.float32),
        pltpu.SemaphoreType.DMA,
        pltpu.SemaphoreType.DMA,
    ]
    out = jax.ShapeDtypeStruct((2 * NP, 16), jnp.float32)

    def body(ones_hbm, z_hbm, bldst, st_hbm, nb_hbm, out_hbm,
             lvm0, lvm1, rows, zstage, stv, nbv, acc, siL0, siL1):
        w = 2 * lax.axis_index("s") + lax.axis_index("c")
        pltpu.sync_copy(st_hbm.at[pl.ds(w * 16, 16)], stv)
        pltpu.sync_copy(nb_hbm.at[pl.ds(w * 16, 16)], nbv)
        pltpu.sync_copy(ones_hbm, rows)
        pltpu.sync_copy(z_hbm, zstage)
        _agg_inner(None, None, bldst, out_hbm, lvm0, lvm1, None, None,
                   rows, None, zstage, stv, nbv, acc,
                   siL0, siL1, None, None, None, None, True, BATCH)

    return pl.kernel(body, out_type=out, mesh=_mesh(),
                     scratch_types=scratch, compiler_params=_SC_PARAMS)


def _prep_body(d0_ref, d1_ref, x_ref, dinv_ref, g_ref):
    a = d0_ref[...] + d1_ref[...]
    dinv = lax.rsqrt(a[:, 0:1] + 1.0)
    dinv_ref[...] = dinv
    g_ref[...] = x_ref[...] * dinv


_prep_call = pl.pallas_call(
    _prep_body,
    grid=(NP // 512,),
    in_specs=[pl.BlockSpec((512, 16), lambda i: (i, 0)),
              pl.BlockSpec((512, 16), lambda i: (i + NP // 512, 0)),
              pl.BlockSpec((512, 16), lambda i: (i, 0))],
    out_specs=[pl.BlockSpec((512, 1), lambda i: (i, 0)),
               pl.BlockSpec((512, 16), lambda i: (i, 0))],
    out_shape=[jax.ShapeDtypeStruct((NP, 1), jnp.float32),
               jax.ShapeDtypeStruct((NP, 16), jnp.float32)],
)


def _make_mm(w_in, w_out, last):
    def body(a0_ref, a1_ref, dinv_ref, w_ref, b_ref, o_ref):
        dinv = dinv_ref[...]
        y = (a0_ref[...] + a1_ref[...]) * dinv
        h = jnp.dot(y, w_ref[...], preferred_element_type=jnp.float32)
        h = jnp.maximum(h + b_ref[...], 0.0)
        o_ref[...] = h if last else h * dinv

    return pl.pallas_call(
        body,
        grid=(NP // 512,),
        in_specs=[pl.BlockSpec((512, w_in), lambda i: (i, 0)),
                  pl.BlockSpec((512, w_in), lambda i: (i + NP // 512, 0)),
                  pl.BlockSpec((512, 1), lambda i: (i, 0)),
                  pl.BlockSpec((w_in, w_out), lambda i: (0, 0)),
                  pl.BlockSpec((1, w_out), lambda i: (0, 0))],
        out_specs=pl.BlockSpec((512, w_out), lambda i: (i, 0)),
        out_shape=jax.ShapeDtypeStruct((NP, w_out), jnp.float32),
    )


_mm_calls = {(16, 32): _make_mm(16, 32, False),
             (32, 64): _make_mm(32, 64, False),
             (64, 128): _make_mm(64, 128, False),
             (128, 256): _make_mm(128, 256, True)}


def _pool_body(h_ref, b_ref, wf_ref, bf_ref, o_ref, sum_acc, cnt_acc):
    i = pl.program_id(0)

    @pl.when(i == 0)
    def _():
        sum_acc[...] = jnp.zeros_like(sum_acc)
        cnt_acc[...] = jnp.zeros_like(cnt_acc)

    oh = (b_ref[...] == lax.broadcasted_iota(jnp.int32, (1, G), 1))
    oh = oh.astype(jnp.float32)
    dn = (((0,), (0,)), ((), ()))
    sum_acc[...] += lax.dot_general(oh, h_ref[...], dn,
                                    preferred_element_type=jnp.float32)
    ones = jnp.ones((512, 1), jnp.float32)
    cnt_acc[...] += lax.dot_general(oh, ones, dn,
                                    preferred_element_type=jnp.float32)

    @pl.when(i == pl.num_programs(0) - 1)
    def _():
        pooled = sum_acc[...] / jnp.maximum(cnt_acc[...], 1.0)
        logits = jnp.dot(pooled, wf_ref[...],
                         preferred_element_type=jnp.float32) + bf_ref[...]
        m = jnp.max(logits, axis=-1, keepdims=True)
        e = jnp.exp(logits - m)
        lse = jnp.log(jnp.sum(e, axis=-1, keepdims=True)) + m
        o_ref[...] = logits - lse


_pool_call = pl.pallas_call(
    _pool_body,
    grid=(NP // 512,),
    in_specs=[pl.BlockSpec((512, 256), lambda i: (i, 0)),
              pl.BlockSpec((512, 1), lambda i: (i, 0)),
              pl.BlockSpec((256, NCLS), lambda i: (0, 0)),
              pl.BlockSpec((1, NCLS), lambda i: (0, 0))],
    out_specs=pl.BlockSpec((G, NCLS), lambda i: (0, 0)),
    out_shape=jax.ShapeDtypeStruct((G, NCLS), jnp.float32),
    scratch_shapes=[pltpu.VMEM((G, 256), jnp.float32),
                    pltpu.VMEM((G, 1), jnp.float32)],
)


def kernel(x, edge_index, batch, W1, b1, W2, b2, W3, b3, W4, b4, Wf, bf):
    src = edge_index[0].astype(jnp.int32)
    dst = edge_index[1].astype(jnp.int32)

    cnts = _count_call()(dst)
    st_flat, nb_flat = _scan_call(cnts.reshape(NW * 16, 1))
    st_tbl = st_flat.reshape(NW * 16)
    nb_tbl = nb_flat.reshape(NW * 16)
    bsrc, bldst = _bucket_call()(src, dst, st_tbl)

    ones_rows = jnp.zeros((BATCH, 16), jnp.float32).at[:, 0].set(1.0)
    z16 = jnp.zeros((ZR, 16), jnp.float32)
    dacc = _make_deg()(ones_rows, z16, bldst, st_tbl, nb_tbl)
    xpad = jnp.pad(x, ((0, NP - N), (0, 16 - 5)))
    dinv, g = _prep_call(dacc, dacc, xpad)

    W1p = jnp.pad(W1, ((0, 16 - 5), (0, 0)))
    layers = [(W1p, b1, 16, 32), (W2, b2, 32, 64),
              (W3, b3, 64, 128), (W4, b4, 128, 256)]
    for Wl, bl, wi, wo in layers:
        zw = z16 if wi == 16 else jnp.zeros((ZR, wi), jnp.float32)
        accl = _make_agg(wi)(g, zw, bsrc, bldst, st_tbl, nb_tbl)
        g = _mm_calls[(wi, wo)](accl, accl, dinv, Wl, bl.reshape(1, wo))

    bpad = jnp.pad(batch.astype(jnp.int32), (0, NP - N),
                   constant_values=-1).reshape(NP, 1)
    return _pool_call(g, bpad, Wf, bf.reshape(1, NCLS))


# speedup vs baseline: 1.3953x; 1.3953x over previous
import functools

import jax
import jax.numpy as jnp
from jax import lax
from jax.experimental import pallas as pl
from jax.experimental.pallas import tpu as pltpu
from jax.experimental.pallas import tpu_sc as plsc

N = 100000
E = 1600000
G = 64
NCLS = 10
RP = 8192
P = 13
NP = P * RP
NW = 32
CHUNK = E // NW
SUBA = 25088
SUBB = CHUNK - SUBA
BATCH = 256
E8 = E + NW * P * BATCH
ACCR = RP + 8
TROWS = RP // 16
ZR = 64

_SC_PARAMS = pltpu.CompilerParams(needs_layout_passes=False,
                                  use_tc_tiling_on_sc=False)


@functools.cache
def _mesh():
    return plsc.VectorSubcoreMesh(core_axis_name="c", subcore_axis_name="s",
                                  num_cores=2, num_subcores=16)


def _count_body(dst_hbm, cnt_hbm, dvm, cvm):
    c = lax.axis_index("c")
    s = lax.axis_index("s")
    w = 2 * s + c
    pltpu.sync_copy(dst_hbm.at[pl.ds(w * CHUNK, CHUNK)], dvm)
    lanes = lax.iota(jnp.int32, 16)

    def body(i, accs):
        d = dvm[pl.ds(i * 16, 16)]
        pid = lax.shift_right_logical(d, 13)
        return tuple(a + (pid == p).astype(jnp.int32)
                     for p, a in enumerate(accs))

    accs = lax.fori_loop(0, CHUNK // 16, body,
                         tuple(jnp.zeros((16,), jnp.int32)
                               for _ in range(P)))
    cvec = jnp.zeros((16,), jnp.int32)
    for p in range(P):
        cvec = jnp.where(lanes == p, jnp.sum(accs[p]), cvec)
    cvm[...] = cvec
    pltpu.sync_copy(cvm, cnt_hbm.at[pl.ds(w * 16, 16)])


@functools.cache
def _count_call():
    return pl.kernel(
        _count_body,
        out_type=jax.ShapeDtypeStruct((NW * 16,), jnp.int32),
        mesh=_mesh(),
        scratch_types=[pltpu.VMEM((CHUNK,), jnp.int32),
                       pltpu.VMEM((16,), jnp.int32)],
        compiler_params=_SC_PARAMS,
    )


def _scan_body(cnt_ref, st_ref, nb_ref):
    nbi = lax.shift_right_logical(cnt_ref[...] + (BATCH - 1), 8)
    nbf = nbi.astype(jnp.float32)
    r = lax.broadcasted_iota(jnp.int32, (NW * 16, NW * 16), 0)
    col = lax.broadcasted_iota(jnp.int32, (NW * 16, NW * 16), 1)
    tril = (col < r).astype(jnp.float32)
    st = jnp.dot(tril, nbf, preferred_element_type=jnp.float32)
    st_ref[...] = st.astype(jnp.int32)
    nb_ref[...] = nbi


_scan_call = pl.pallas_call(
    _scan_body,
    out_shape=[jax.ShapeDtypeStruct((NW * 16, 1), jnp.int32),
               jax.ShapeDtypeStruct((NW * 16, 1), jnp.int32)],
)


def _bucket_body(src_hbm, dst_hbm, st_hbm, bsrc_hbm, bldst_hbm,
                 svm, dvm, bs, bl, stv, cur_ref, nfl_ref):
    c = lax.axis_index("c")
    s = lax.axis_index("s")
    w = 2 * s + c
    pltpu.sync_copy(st_hbm.at[pl.ds(w * 16, 16)], stv)
    for p in range(P):
        cur_ref[p] = jnp.int32(0)
        nfl_ref[p] = jnp.int32(0)

    PB = BATCH + 16

    def handle(p, sv, ld, m):
        cur = cur_ref[p]
        ps = plsc.cumsum(jnp.where(m, 1, 0))
        idx = p * PB + cur + ps - 1
        plsc.store_scatter(bs, [idx], sv, mask=m)
        plsc.store_scatter(bl, [idx], ld, mask=m)
        cur = cur + ps[15]
        cur_ref[p] = cur

        @pl.when(cur >= BATCH)
        def _():
            segw = (stv[...][p] + nfl_ref[p]) * BATCH
            pltpu.sync_copy(bs.at[pl.ds(p * PB, BATCH)],
                            bsrc_hbm.at[pl.ds(segw, BATCH)])
            pltpu.sync_copy(bl.at[pl.ds(p * PB, BATCH)],
                            bldst_hbm.at[pl.ds(segw, BATCH)])
            ts = bs[pl.ds(p * PB + BATCH, 16)]
            tl = bl[pl.ds(p * PB + BATCH, 16)]
            bs[pl.ds(p * PB, 16)] = ts
            bl[pl.ds(p * PB, 16)] = tl
            nfl_ref[p] = nfl_ref[p] + 1
            cur_ref[p] = cur - BATCH

    for half, sub in enumerate((SUBA, SUBB)):
        base = w * CHUNK + half * SUBA
        pltpu.sync_copy(src_hbm.at[pl.ds(base, sub)],
                        svm.at[pl.ds(0, sub)])
        pltpu.sync_copy(dst_hbm.at[pl.ds(base, sub)],
                        dvm.at[pl.ds(0, sub)])

        def body(i, _):
            sv = svm[pl.ds(i * 16, 16)]
            dv = dvm[pl.ds(i * 16, 16)]
            pid = lax.shift_right_logical(dv, 13)
            ld = jnp.bitwise_and(dv, RP - 1)
            for p in range(P):
                handle(p, sv, ld, pid == p)
            return 0

        lax.fori_loop(0, sub // 16, body, 0)

    lanes = lax.iota(jnp.int32, 16)
    for p in range(P):
        cur = cur_ref[p]
        for j in range(BATCH // 16):
            fill = (lanes + j * 16) >= cur
            ov_s = bs[pl.ds(p * PB + j * 16, 16)]
            ov_l = bl[pl.ds(p * PB + j * 16, 16)]
            bs[pl.ds(p * PB + j * 16, 16)] = jnp.where(fill, 0, ov_s)
            bl[pl.ds(p * PB + j * 16, 16)] = jnp.where(fill, RP + (j % 8),
                                                       ov_l)

        @pl.when(cur > 0)
        def _():
            segw = (stv[...][p] + nfl_ref[p]) * BATCH
            pltpu.sync_copy(bs.at[pl.ds(p * PB, BATCH)],
                            bsrc_hbm.at[pl.ds(segw, BATCH)])
            pltpu.sync_copy(bl.at[pl.ds(p * PB, BATCH)],
                            bldst_hbm.at[pl.ds(segw, BATCH)])


@functools.cache
def _bucket_call():
    return pl.kernel(
        _bucket_body,
        out_type=(jax.ShapeDtypeStruct((E8,), jnp.int32),
                  jax.ShapeDtypeStruct((E8,), jnp.int32)),
        mesh=_mesh(),
        scratch_types=[
            pltpu.VMEM((SUBA,), jnp.int32),
            pltpu.VMEM((SUBA,), jnp.int32),
            pltpu.VMEM((P * (BATCH + 16),), jnp.int32),
            pltpu.VMEM((P * (BATCH + 16),), jnp.int32),
            pltpu.VMEM((16,), jnp.int32),
            pltpu.SMEM((16,), jnp.int32),
            pltpu.SMEM((16,), jnp.int32),
        ],
        compiler_params=_SC_PARAMS,
    )


def _agg_inner(g_hbm, bsrc, bldst, out_hbm, lvm, svm, rows, zstage,
               stv, nbv, acc, sem, deg_pass):
    c = lax.axis_index("c")
    s = lax.axis_index("s")

    for p in range(P):
        myrows = p * RP + s * TROWS
        if deg_pass:
            for k in range(TROWS // ZR):
                pltpu.sync_copy(zstage,
                                acc.at[pl.ds(s * TROWS + k * ZR, ZR)])
        else:
            @pl.when(c == 0)
            def _():
                pltpu.sync_copy(g_hbm.at[pl.ds(myrows, TROWS)],
                                acc.at[pl.ds(s * TROWS, TROWS)])

            @pl.when(c == 1)
            def _():
                for k in range(TROWS // ZR):
                    pltpu.sync_copy(zstage,
                                    acc.at[pl.ds(s * TROWS + k * ZR, ZR)])

        plsc.subcore_barrier()
        stb = stv[...][p]
        nb = nbv[...][p]

        def eb(j, _):
            off = (stb + j) * BATCH
            pltpu.sync_copy(bldst.at[pl.ds(off, BATCH)], lvm)
            if not deg_pass:
                pltpu.sync_copy(bsrc.at[pl.ds(off, BATCH)], svm)
                pltpu.async_copy(g_hbm.at[svm], rows, sem).wait()
            pltpu.sync_copy(rows, acc.at[lvm], add=True)
            return 0

        lax.fori_loop(0, nb, eb, 0)
        plsc.subcore_barrier()
        pltpu.sync_copy(acc.at[pl.ds(s * TROWS, TROWS)],
                        out_hbm.at[pl.ds(c * NP + myrows, TROWS)])
        plsc.subcore_barrier()


@functools.cache
def _make_agg(w_feat):
    scratch = [
        pltpu.VMEM((BATCH,), jnp.int32),
        pltpu.VMEM((BATCH,), jnp.int32),
        pltpu.VMEM((BATCH, w_feat), jnp.float32),
        pltpu.VMEM((ZR, w_feat), jnp.float32),
        pltpu.VMEM((16,), jnp.int32),
        pltpu.VMEM((16,), jnp.int32),
        pltpu.VMEM_SHARED((ACCR, w_feat), jnp.float32),
        pltpu.SemaphoreType.DMA,
    ]
    out = jax.ShapeDtypeStruct((2 * NP, w_feat), jnp.float32)

    def body(g_hbm, z_hbm, bsrc, bldst, st_hbm, nb_hbm, out_hbm,
             lvm, svm, rows, zstage, stv, nbv, acc, sem):
        w = 2 * lax.axis_index("s") + lax.axis_index("c")
        pltpu.sync_copy(st_hbm.at[pl.ds(w * 16, 16)], stv)
        pltpu.sync_copy(nb_hbm.at[pl.ds(w * 16, 16)], nbv)
        pltpu.sync_copy(z_hbm, zstage)
        _agg_inner(g_hbm, bsrc, bldst, out_hbm, lvm, svm, rows, zstage,
                   stv, nbv, acc, sem, False)

    return pl.kernel(body, out_type=out, mesh=_mesh(),
                     scratch_types=scratch, compiler_params=_SC_PARAMS)


@functools.cache
def _make_deg():
    scratch = [
        pltpu.VMEM((BATCH,), jnp.int32),
        pltpu.VMEM((BATCH, 16), jnp.float32),
        pltpu.VMEM((ZR, 16), jnp.float32),
        pltpu.VMEM((16,), jnp.int32),
        pltpu.VMEM((16,), jnp.int32),
        pltpu.VMEM_SHARED((ACCR, 16), jnp.float32),
        pltpu.SemaphoreType.DMA,
    ]
    out = jax.ShapeDtypeStruct((2 * NP, 16), jnp.float32)

    def body(ones_hbm, z_hbm, bldst, st_hbm, nb_hbm, out_hbm, lvm, rows,
             zstage, stv, nbv, acc, sem):
        w = 2 * lax.axis_index("s") + lax.axis_index("c")
        pltpu.sync_copy(st_hbm.at[pl.ds(w * 16, 16)], stv)
        pltpu.sync_copy(nb_hbm.at[pl.ds(w * 16, 16)], nbv)
        pltpu.sync_copy(ones_hbm, rows)
        pltpu.sync_copy(z_hbm, zstage)
        _agg_inner(None, None, bldst, out_hbm, lvm, None, rows, zstage,
                   stv, nbv, acc, sem, True)

    return pl.kernel(body, out_type=out, mesh=_mesh(),
                     scratch_types=scratch, compiler_params=_SC_PARAMS)


def _prep_body(d0_ref, d1_ref, x_ref, dinv_ref, g_ref):
    a = d0_ref[...] + d1_ref[...]
    dinv = lax.rsqrt(a[:, 0:1] + 1.0)
    dinv_ref[...] = dinv
    g_ref[...] = x_ref[...] * dinv


_prep_call = pl.pallas_call(
    _prep_body,
    grid=(NP // 512,),
    in_specs=[pl.BlockSpec((512, 16), lambda i: (i, 0)),
              pl.BlockSpec((512, 16), lambda i: (i + NP // 512, 0)),
              pl.BlockSpec((512, 16), lambda i: (i, 0))],
    out_specs=[pl.BlockSpec((512, 1), lambda i: (i, 0)),
               pl.BlockSpec((512, 16), lambda i: (i, 0))],
    out_shape=[jax.ShapeDtypeStruct((NP, 1), jnp.float32),
               jax.ShapeDtypeStruct((NP, 16), jnp.float32)],
)


def _make_mm(w_in, w_out, last):
    def body(a0_ref, a1_ref, dinv_ref, w_ref, b_ref, o_ref):
        dinv = dinv_ref[...]
        y = (a0_ref[...] + a1_ref[...]) * dinv
        h = jnp.dot(y, w_ref[...], preferred_element_type=jnp.float32)
        h = jnp.maximum(h + b_ref[...], 0.0)
        o_ref[...] = h if last else h * dinv

    return pl.pallas_call(
        body,
        grid=(NP // 512,),
        in_specs=[pl.BlockSpec((512, w_in), lambda i: (i, 0)),
                  pl.BlockSpec((512, w_in), lambda i: (i + NP // 512, 0)),
                  pl.BlockSpec((512, 1), lambda i: (i, 0)),
                  pl.BlockSpec((w_in, w_out), lambda i: (0, 0)),
                  pl.BlockSpec((1, w_out), lambda i: (0, 0))],
        out_specs=pl.BlockSpec((512, w_out), lambda i: (i, 0)),
        out_shape=jax.ShapeDtypeStruct((NP, w_out), jnp.float32),
    )


_mm_calls = {(16, 32): _make_mm(16, 32, False),
             (32, 64): _make_mm(32, 64, False),
             (64, 128): _make_mm(64, 128, False),
             (128, 256): _make_mm(128, 256, True)}


def _pool_body(h_ref, b_ref, wf_ref, bf_ref, o_ref, sum_acc, cnt_acc):
    i = pl.program_id(0)

    @pl.when(i == 0)
    def _():
        sum_acc[...] = jnp.zeros_like(sum_acc)
        cnt_acc[...] = jnp.zeros_like(cnt_acc)

    oh = (b_ref[...] == lax.broadcasted_iota(jnp.int32, (1, G), 1))
    oh = oh.astype(jnp.float32)
    dn = (((0,), (0,)), ((), ()))
    sum_acc[...] += lax.dot_general(oh, h_ref[...], dn,
                                    preferred_element_type=jnp.float32)
    ones = jnp.ones((512, 1), jnp.float32)
    cnt_acc[...] += lax.dot_general(oh, ones, dn,
                                    preferred_element_type=jnp.float32)

    @pl.when(i == pl.num_programs(0) - 1)
    def _():
        pooled = sum_acc[...] / jnp.maximum(cnt_acc[...], 1.0)
        logits = jnp.dot(pooled, wf_ref[...],
                         preferred_element_type=jnp.float32) + bf_ref[...]
        m = jnp.max(logits, axis=-1, keepdims=True)
        e = jnp.exp(logits - m)
        lse = jnp.log(jnp.sum(e, axis=-1, keepdims=True)) + m
        o_ref[...] = logits - lse


_pool_call = pl.pallas_call(
    _pool_body,
    grid=(NP // 512,),
    in_specs=[pl.BlockSpec((512, 256), lambda i: (i, 0)),
              pl.BlockSpec((512, 1), lambda i: (i, 0)),
              pl.BlockSpec((256, NCLS), lambda i: (0, 0)),
              pl.BlockSpec((1, NCLS), lambda i: (0, 0))],
    out_specs=pl.BlockSpec((G, NCLS), lambda i: (0, 0)),
    out_shape=jax.ShapeDtypeStruct((G, NCLS), jnp.float32),
    scratch_shapes=[pltpu.VMEM((G, 256), jnp.float32),
                    pltpu.VMEM((G, 1), jnp.float32)],
)


def kernel(x, edge_index, batch, W1, b1, W2, b2, W3, b3, W4, b4, Wf, bf):
    src = edge_index[0].astype(jnp.int32)
    dst = edge_index[1].astype(jnp.int32)

    cnts = _count_call()(dst)
    st_flat, nb_flat = _scan_call(cnts.reshape(NW * 16, 1))
    st_tbl = st_flat.reshape(NW * 16)
    nb_tbl = nb_flat.reshape(NW * 16)
    bsrc, bldst = _bucket_call()(src, dst, st_tbl)

    ones_rows = jnp.zeros((BATCH, 16), jnp.float32).at[:, 0].set(1.0)
    z16 = jnp.zeros((ZR, 16), jnp.float32)
    dacc = _make_deg()(ones_rows, z16, bldst, st_tbl, nb_tbl)
    xpad = jnp.pad(x, ((0, NP - N), (0, 16 - 5)))
    dinv, g = _prep_call(dacc, dacc, xpad)

    W1p = jnp.pad(W1, ((0, 16 - 5), (0, 0)))
    layers = [(W1p, b1, 16, 32), (W2, b2, 32, 64),
              (W3, b3, 64, 128), (W4, b4, 128, 256)]
    for Wl, bl, wi, wo in layers:
        zw = z16 if wi == 16 else jnp.zeros((ZR, wi), jnp.float32)
        accl = _make_agg(wi)(g, zw, bsrc, bldst, st_tbl, nb_tbl)
        g = _mm_calls[(wi, wo)](accl, accl, dinv, Wl, bl.reshape(1, wo))

    bpad = jnp.pad(batch.astype(jnp.int32), (0, NP - N),
                   constant_values=-1).reshape(NP, 1)
    return _pool_call(g, bpad, Wf, bf.reshape(1, NCLS))
